# Initial kernel scaffold; baseline (speedup 1.0000x reference)
import functools

import jax
import jax.numpy as jnp
from jax import lax
from jax.experimental import pallas as pl
from jax.experimental.pallas import tpu as pltpu
from jax.experimental.pallas import tpu_sc as plsc

N = 100000
EMBED_DIM = 16
MACE_DIM = 4
OUT_DIM = EMBED_DIM + MACE_DIM
VOCAB = 100

NC, NS = 2, 16
NW = NC * NS
CHUNK = 1024
MAIN_CHUNKS = 3
MAIN_ROWS = NW * MAIN_CHUNKS * CHUNK
TAIL = N - MAIN_ROWS
TAIL_FULL = TAIL // 128
TAIL_REM = TAIL - TAIL_FULL * 128
GATHER_BURST = 128


def _process(table, zref, ntref, saref, out, idx_v, nt_v, sa_v, out_v, sem,
             base, rows):
    pltpu.sync_copy(zref.at[pl.ds(base, rows)], idx_v.at[pl.ds(0, rows)])
    copies = []
    for j in range(0, rows, GATHER_BURST):
        g = min(GATHER_BURST, rows - j)
        copies.append(pltpu.async_copy(
            table.at[idx_v.at[pl.ds(j, g)]], out_v.at[pl.ds(j, g)], sem))
    pltpu.sync_copy(ntref.at[pl.ds(base, rows)], nt_v.at[pl.ds(0, rows)])
    pltpu.sync_copy(saref.at[pl.ds(base, rows)], sa_v.at[pl.ds(0, rows)])
    for c in copies:
        c.wait()

    lane = lax.iota(jnp.int32, 16)
    rowpat = lane >> 2
    colpat = lane & 3

    def body(i, carry):
        r = rowpat + i * 4
        nt = plsc.load_gather(nt_v, [r])
        sa = plsc.load_gather(sa_v, [r, colpat])
        plsc.addupdate_scatter(out_v, [r, colpat + EMBED_DIM], sa,
                               mask=(nt == 1))
        return carry

    lax.fori_loop(0, rows // 4, body, 0)
    pltpu.sync_copy(out_v.at[pl.ds(0, rows)], out.at[pl.ds(base, rows)])


@functools.partial(
    pl.kernel,
    out_type=jax.ShapeDtypeStruct((N, OUT_DIM), jnp.float32),
    mesh=plsc.VectorSubcoreMesh(core_axis_name="c", subcore_axis_name="s"),
    scratch_types=[
        pltpu.VMEM((CHUNK,), jnp.int32),
        pltpu.VMEM((CHUNK,), jnp.int32),
        pltpu.VMEM((CHUNK, MACE_DIM), jnp.float32),
        pltpu.VMEM((CHUNK, OUT_DIM), jnp.float32),
        pltpu.SemaphoreType.DMA,
    ],
)
def _feature_builder(table, zref, ntref, saref, out,
                     idx_v, nt_v, sa_v, out_v, sem):
    w = lax.axis_index("s") * NC + lax.axis_index("c")
    for j in range(MAIN_CHUNKS):
        base = (j * NW + w) * CHUNK
        _process(table, zref, ntref, saref, out,
                 idx_v, nt_v, sa_v, out_v, sem, base, CHUNK)

    @pl.when(w < TAIL_FULL)
    def _():
        _process(table, zref, ntref, saref, out,
                 idx_v, nt_v, sa_v, out_v, sem, MAIN_ROWS + w * 128, 128)

    @pl.when(w == TAIL_FULL)
    def _():
        _process(table, zref, ntref, saref, out,
                 idx_v, nt_v, sa_v, out_v, sem,
                 MAIN_ROWS + TAIL_FULL * 128, TAIL_REM)


def kernel(z, node_type, site_ads, z_embed):
    table = jnp.concatenate(
        [z_embed, jnp.zeros((VOCAB, MACE_DIM), jnp.float32)], axis=1)
    return _feature_builder(table, z.astype(jnp.int32),
                            node_type.astype(jnp.int32), site_ads)


# speedup vs baseline: 1.2409x; 1.2409x over previous
import functools

import jax
import jax.numpy as jnp
from jax import lax
from jax.experimental import pallas as pl
from jax.experimental.pallas import tpu as pltpu
from jax.experimental.pallas import tpu_sc as plsc

N = 100000
EMBED_DIM = 16
MACE_DIM = 4
OUT_DIM = EMBED_DIM + MACE_DIM
PAD_DIM = 24
VOCAB = 100

NC, NS = 2, 16
NW = NC * NS
CHUNK = 1024
MAIN_CHUNKS = 3
MAIN_ROWS = NW * MAIN_CHUNKS * CHUNK
TAIL = N - MAIN_ROWS
TAIL_FULL = TAIL // 128
TAIL_REM = TAIL - TAIL_FULL * 128
GATHER_BURST = 128


def _process(table, zref, ntref, saref, out, idx_v, nt_v, sa_v, out_v, sem,
             base, rows):
    pltpu.sync_copy(zref.at[pl.ds(base, rows)], idx_v.at[pl.ds(0, rows)])
    copies = []
    for j in range(0, rows, GATHER_BURST):
        g = min(GATHER_BURST, rows - j)
        copies.append(pltpu.async_copy(
            table.at[idx_v.at[pl.ds(j, g)]], out_v.at[pl.ds(j, g)], sem))
    pltpu.sync_copy(ntref.at[pl.ds(base, rows)], nt_v.at[pl.ds(0, rows)])
    pltpu.sync_copy(saref.at[pl.ds(base * MACE_DIM, rows * MACE_DIM)],
                    sa_v.at[pl.ds(0, rows * MACE_DIM)])
    for c in copies:
        c.wait()

    lane = lax.iota(jnp.int32, 16)
    rowpat = lane >> 2
    colpat = (lane & 3) + EMBED_DIM

    def body(i, carry):
        r = rowpat + i * 4
        nt = plsc.load_gather(nt_v, [r])
        sa = sa_v[pl.ds(i * 16, 16)]
        plsc.store_scatter(out_v, [r, colpat], sa, mask=(nt == 1))
        return carry

    lax.fori_loop(0, rows // 4, body, 0)
    pltpu.sync_copy(out_v.at[pl.ds(0, rows)], out.at[pl.ds(base, rows)])


@functools.partial(
    pl.kernel,
    out_type=jax.ShapeDtypeStruct((N, PAD_DIM), jnp.float32),
    mesh=plsc.VectorSubcoreMesh(core_axis_name="c", subcore_axis_name="s"),
    scratch_types=[
        pltpu.VMEM((CHUNK,), jnp.int32),
        pltpu.VMEM((CHUNK,), jnp.int32),
        pltpu.VMEM((CHUNK * MACE_DIM,), jnp.float32),
        pltpu.VMEM((CHUNK, PAD_DIM), jnp.float32),
        pltpu.SemaphoreType.DMA,
    ],
    compiler_params=pltpu.CompilerParams(
        needs_layout_passes=False, use_tc_tiling_on_sc=False),
)
def _feature_builder(table, zref, ntref, saref, out,
                     idx_v, nt_v, sa_v, out_v, sem):
    w = lax.axis_index("s") * NC + lax.axis_index("c")
    for j in range(MAIN_CHUNKS):
        base = (j * NW + w) * CHUNK
        _process(table, zref, ntref, saref, out,
                 idx_v, nt_v, sa_v, out_v, sem, base, CHUNK)

    @pl.when(w < TAIL_FULL)
    def _():
        _process(table, zref, ntref, saref, out,
                 idx_v, nt_v, sa_v, out_v, sem, MAIN_ROWS + w * 128, 128)

    @pl.when(w == TAIL_FULL)
    def _():
        _process(table, zref, ntref, saref, out,
                 idx_v, nt_v, sa_v, out_v, sem,
                 MAIN_ROWS + TAIL_FULL * 128, TAIL_REM)


def kernel(z, node_type, site_ads, z_embed):
    table = jnp.concatenate(
        [z_embed, jnp.zeros((VOCAB, PAD_DIM - EMBED_DIM), jnp.float32)],
        axis=1)
    out24 = _feature_builder(table, z.astype(jnp.int32),
                             node_type.astype(jnp.int32),
                             site_ads.reshape(-1))
    return out24[:, :OUT_DIM]


# speedup vs baseline: 1.5220x; 1.2265x over previous
import functools

import jax
import jax.numpy as jnp
from jax import lax
from jax.experimental import pallas as pl
from jax.experimental.pallas import tpu as pltpu
from jax.experimental.pallas import tpu_sc as plsc

N = 100000
EMBED_DIM = 16
MACE_DIM = 4
OUT_DIM = EMBED_DIM + MACE_DIM
PAD_DIM = 24
VOCAB = 100

NC, NS = 2, 16
NW = NC * NS
BIG = 3128
SMALL = 3120
NBIG = 20
GATHER_BURST = 128


def _process(table_sh, zref, ntref, saref, out, idx_v, nt_v, sa_v, rows_v,
             sem, base, rows):
    pltpu.sync_copy(zref.at[pl.ds(base, rows)], idx_v.at[pl.ds(0, rows)])
    copies = []
    for j in range(0, rows, GATHER_BURST):
        g = min(GATHER_BURST, rows - j)
        copies.append(pltpu.async_copy(
            table_sh.at[idx_v.at[pl.ds(j, g)]], rows_v.at[pl.ds(j, g)], sem))
    pltpu.sync_copy(ntref.at[pl.ds(base, rows)], nt_v.at[pl.ds(0, rows)])
    pltpu.sync_copy(saref.at[pl.ds(base * MACE_DIM, rows * MACE_DIM)],
                    sa_v.at[pl.ds(0, rows * MACE_DIM)])
    for c in copies:
        c.wait()

    lane = lax.iota(jnp.int32, 16)
    rowpat = lane >> 2
    colpat = (lane & 3) + EMBED_DIM

    def body(i, carry):
        r = rowpat + i * 4
        nt = plsc.load_gather(nt_v, [r])
        sa = sa_v[pl.ds(i * 16, 16)]
        plsc.store_scatter(rows_v, [r, colpat], sa, mask=(nt == 1))
        return carry

    lax.fori_loop(0, rows // 4, body, 0)
    pltpu.sync_copy(rows_v.at[pl.ds(0, rows)], out.at[pl.ds(base, rows)])


@functools.partial(
    pl.kernel,
    out_type=jax.ShapeDtypeStruct((N, PAD_DIM), jnp.float32),
    mesh=plsc.VectorSubcoreMesh(core_axis_name="c", subcore_axis_name="s"),
    scratch_types=[
        pltpu.VMEM((BIG,), jnp.int32),
        pltpu.VMEM((BIG,), jnp.int32),
        pltpu.VMEM((BIG * MACE_DIM,), jnp.float32),
        pltpu.VMEM((BIG, PAD_DIM), jnp.float32),
        pltpu.VMEM_SHARED((VOCAB, PAD_DIM), jnp.float32),
        pltpu.SemaphoreType.DMA,
    ],
    compiler_params=pltpu.CompilerParams(
        needs_layout_passes=False, use_tc_tiling_on_sc=False),
)
def _feature_builder(table, zref, ntref, saref, out,
                     idx_v, nt_v, sa_v, rows_v, table_sh, sem):
    sid = lax.axis_index("s")
    w = sid * NC + lax.axis_index("c")

    @pl.when(sid == 0)
    def _():
        pltpu.sync_copy(table, table_sh)

    plsc.subcore_barrier()

    @pl.when(w < NBIG)
    def _():
        _process(table_sh, zref, ntref, saref, out,
                 idx_v, nt_v, sa_v, rows_v, sem, w * BIG, BIG)

    @pl.when(w >= NBIG)
    def _():
        _process(table_sh, zref, ntref, saref, out,
                 idx_v, nt_v, sa_v, rows_v, sem,
                 NBIG * BIG + (w - NBIG) * SMALL, SMALL)


def kernel(z, node_type, site_ads, z_embed):
    table = jnp.concatenate(
        [z_embed, jnp.zeros((VOCAB, PAD_DIM - EMBED_DIM), jnp.float32)],
        axis=1)
    out24 = _feature_builder(table, z.astype(jnp.int32),
                             node_type.astype(jnp.int32),
                             site_ads.reshape(-1))
    return out24[:, :OUT_DIM]
